# baseline (speedup 1.0000x reference)
import functools

import jax
import jax.numpy as jnp
from jax.experimental import pallas as pl
from jax.experimental.pallas import tpu as pltpu

N = 10000
E = 320000
HEADS = 8


def _mm_body(x_ref, w_ref, o_ref):
    o_ref[...] = jnp.dot(x_ref[...], w_ref[...],
                         preferred_element_type=jnp.float32)


def _matmul(x, w, bm=1000):
    m, k = x.shape
    k2, n = w.shape
    grid = (m // bm,)
    return pl.pallas_call(
        _mm_body,
        grid=grid,
        in_specs=[
            pl.BlockSpec((bm, k), lambda i: (i, 0)),
            pl.BlockSpec((k, n), lambda i: (0, 0)),
        ],
        out_specs=pl.BlockSpec((bm, n), lambda i: (i, 0)),
        out_shape=jax.ShapeDtypeStruct((m, n), jnp.float32),
    )(x, w)


def _conv(x, src, dst, W, att_l, att_r, bias, heads, C):
    n = x.shape[0]
    xl = _matmul(x, W).reshape(n, heads, C)
    x_j = xl[src]
    x_i = xl[dst]
    logits = jnp.sum(x_i * x_j, axis=-1)
    alpha = jnp.sum(x_j * att_l, axis=-1) + jnp.sum(x_i * att_r, axis=-1)
    alpha = alpha * jax.nn.sigmoid(logits)
    alpha = jax.nn.leaky_relu(alpha, negative_slope=0.2)
    amax = jax.ops.segment_max(alpha, dst, num_segments=n)
    ex = jnp.exp(alpha - amax[dst])
    denom = jax.ops.segment_sum(ex, dst, num_segments=n)
    alpha = ex / (denom[dst] + 1e-16)
    out = jax.ops.segment_sum(x_j * alpha[..., None], dst, num_segments=n)
    return out.reshape(n, heads * C) + bias


def kernel(x, edge_index, W1, att_l1, att_r1, bias1, W2, att_l2, att_r2, bias2):
    loop = jnp.arange(N, dtype=edge_index.dtype)
    src = jnp.concatenate([edge_index[0], loop])
    dst = jnp.concatenate([edge_index[1], loop])
    h = jax.nn.elu(_conv(x, src, dst, W1, att_l1, att_r1, bias1, HEADS, 256))
    h = _conv(h, src, dst, W2, att_l2, att_r2, bias2, HEADS, 128)
    return jax.nn.log_softmax(h, axis=1)


# speedup vs baseline: 4.6295x; 4.6295x over previous
import jax
import jax.numpy as jnp
from jax.experimental import pallas as pl

N = 10000
E = 320000
HEADS = 8


def _mm_body(x_ref, w_ref, o_ref):
    o_ref[...] = jnp.dot(x_ref[...], w_ref[...],
                         preferred_element_type=jnp.float32)


def _matmul(x, w, bm=1000):
    m, k = x.shape
    k2, n = w.shape
    grid = (m // bm,)
    return pl.pallas_call(
        _mm_body,
        grid=grid,
        in_specs=[
            pl.BlockSpec((bm, k), lambda i: (i, 0)),
            pl.BlockSpec((k, n), lambda i: (0, 0)),
        ],
        out_specs=pl.BlockSpec((bm, n), lambda i: (i, 0)),
        out_shape=jax.ShapeDtypeStruct((m, n), jnp.float32),
    )(x, w)


def _edge_body(xj_ref, xi_ref, s_ref, al_ref, ar_ref, st_ref, ex_ref, msg_ref):
    xj = xj_ref[...]
    xi = xi_ref[...]
    logits = jnp.dot(xj * xi, s_ref[...], preferred_element_type=jnp.float32)
    al = jnp.dot(xj, al_ref[...], preferred_element_type=jnp.float32)
    ar = jnp.dot(xi, ar_ref[...], preferred_element_type=jnp.float32)
    a = (al + ar) * jax.nn.sigmoid(logits)
    alpha = jnp.where(a > 0, a, 0.2 * a)
    ex = jnp.exp(alpha)
    ex_ref[...] = ex
    msg_ref[...] = xj * jnp.dot(ex, st_ref[...],
                                preferred_element_type=jnp.float32)


def _edge_pass(xj, xi, s, al, ar, be=528):
    ne, hc = xj.shape
    grid = (ne // be,)
    return pl.pallas_call(
        _edge_body,
        grid=grid,
        in_specs=[
            pl.BlockSpec((be, hc), lambda i: (i, 0)),
            pl.BlockSpec((be, hc), lambda i: (i, 0)),
            pl.BlockSpec((hc, HEADS), lambda i: (0, 0)),
            pl.BlockSpec((hc, HEADS), lambda i: (0, 0)),
            pl.BlockSpec((hc, HEADS), lambda i: (0, 0)),
            pl.BlockSpec((HEADS, hc), lambda i: (0, 0)),
        ],
        out_specs=[
            pl.BlockSpec((be, HEADS), lambda i: (i, 0)),
            pl.BlockSpec((be, hc), lambda i: (i, 0)),
        ],
        out_shape=[
            jax.ShapeDtypeStruct((ne, HEADS), jnp.float32),
            jax.ShapeDtypeStruct((ne, hc), jnp.float32),
        ],
    )(xj, xi, s, al, ar, s.T)


def _epi1_body(raw_ref, den_ref, st_ref, b_ref, o_ref):
    denb = jnp.dot(den_ref[...], st_ref[...],
                   preferred_element_type=jnp.float32)
    h = raw_ref[...] / (denb + 1e-16) + b_ref[...]
    o_ref[...] = jnp.where(h > 0, h, jnp.exp(jnp.minimum(h, 0.0)) - 1.0)


def _epi2_body(raw_ref, den_ref, st_ref, b_ref, o_ref):
    denb = jnp.dot(den_ref[...], st_ref[...],
                   preferred_element_type=jnp.float32)
    h = raw_ref[...] / (denb + 1e-16) + b_ref[...]
    m = jnp.max(h, axis=1, keepdims=True)
    o_ref[...] = h - m - jnp.log(jnp.sum(jnp.exp(h - m), axis=1,
                                         keepdims=True))


def _epilogue(body, raw, den, s, bias, bn=400):
    n, hc = raw.shape
    grid = (n // bn,)
    return pl.pallas_call(
        body,
        grid=grid,
        in_specs=[
            pl.BlockSpec((bn, hc), lambda i: (i, 0)),
            pl.BlockSpec((bn, HEADS), lambda i: (i, 0)),
            pl.BlockSpec((HEADS, hc), lambda i: (0, 0)),
            pl.BlockSpec((1, hc), lambda i: (0, 0)),
        ],
        out_specs=pl.BlockSpec((bn, hc), lambda i: (i, 0)),
        out_shape=jax.ShapeDtypeStruct((n, hc), jnp.float32),
    )(raw, den, s.T, bias.reshape(1, hc))


def _conv(x, src, dst, W, att_l, att_r, bias, C, epi_body):
    n = x.shape[0]
    hc = HEADS * C
    s = jnp.repeat(jnp.eye(HEADS, dtype=jnp.float32), C, axis=0)
    al_mat = s * att_l.reshape(hc, 1)
    ar_mat = s * att_r.reshape(hc, 1)
    xl = _matmul(x, W)
    ex, msg = _edge_pass(xl[src], xl[dst], s, al_mat, ar_mat)
    denom = jax.ops.segment_sum(ex, dst, num_segments=n)
    raw = jax.ops.segment_sum(msg, dst, num_segments=n)
    return _epilogue(epi_body, raw, denom, s, bias)


def kernel(x, edge_index, W1, att_l1, att_r1, bias1, W2, att_l2, att_r2, bias2):
    loop = jnp.arange(N, dtype=edge_index.dtype)
    src = jnp.concatenate([edge_index[0], loop])
    dst = jnp.concatenate([edge_index[1], loop])
    h = _conv(x, src, dst, W1, att_l1, att_r1, bias1, 256, _epi1_body)
    return _conv(h, src, dst, W2, att_l2, att_r2, bias2, 128, _epi2_body)


# speedup vs baseline: 4.6835x; 1.0116x over previous
import jax
import jax.numpy as jnp
from jax.experimental import pallas as pl
from jax.experimental.pallas import tpu as pltpu

_PAR = pltpu.CompilerParams(dimension_semantics=("parallel",))

N = 10000
E = 320000
HEADS = 8


def _mm_body(x_ref, w_ref, o_ref):
    o_ref[...] = jnp.dot(x_ref[...], w_ref[...],
                         preferred_element_type=jnp.float32)


def _matmul(x, w, bm=1000):
    m, k = x.shape
    k2, n = w.shape
    grid = (m // bm,)
    return pl.pallas_call(
        _mm_body,
        grid=grid,
        in_specs=[
            pl.BlockSpec((bm, k), lambda i: (i, 0)),
            pl.BlockSpec((k, n), lambda i: (0, 0)),
        ],
        out_specs=pl.BlockSpec((bm, n), lambda i: (i, 0)),
        out_shape=jax.ShapeDtypeStruct((m, n), jnp.float32),
        compiler_params=_PAR,
    )(x, w)


def _edge_body(xj_ref, xi_ref, s_ref, al_ref, ar_ref, st_ref, ex_ref, msg_ref):
    xj = xj_ref[...]
    xi = xi_ref[...]
    logits = jnp.dot(xj * xi, s_ref[...], preferred_element_type=jnp.float32)
    al = jnp.dot(xj, al_ref[...], preferred_element_type=jnp.float32)
    ar = jnp.dot(xi, ar_ref[...], preferred_element_type=jnp.float32)
    a = (al + ar) * jax.nn.sigmoid(logits)
    alpha = jnp.where(a > 0, a, 0.2 * a)
    ex = jnp.exp(alpha)
    ex_ref[...] = ex
    msg_ref[...] = xj * jnp.dot(ex, st_ref[...],
                                preferred_element_type=jnp.float32)


def _edge_pass(xj, xi, s, al, ar, be=1000):
    ne, hc = xj.shape
    grid = (ne // be,)
    return pl.pallas_call(
        _edge_body,
        grid=grid,
        in_specs=[
            pl.BlockSpec((be, hc), lambda i: (i, 0)),
            pl.BlockSpec((be, hc), lambda i: (i, 0)),
            pl.BlockSpec((hc, HEADS), lambda i: (0, 0)),
            pl.BlockSpec((hc, HEADS), lambda i: (0, 0)),
            pl.BlockSpec((hc, HEADS), lambda i: (0, 0)),
            pl.BlockSpec((HEADS, hc), lambda i: (0, 0)),
        ],
        out_specs=[
            pl.BlockSpec((be, HEADS), lambda i: (i, 0)),
            pl.BlockSpec((be, hc), lambda i: (i, 0)),
        ],
        out_shape=[
            jax.ShapeDtypeStruct((ne, HEADS), jnp.float32),
            jax.ShapeDtypeStruct((ne, hc), jnp.float32),
        ],
        compiler_params=_PAR,
    )(xj, xi, s, al, ar, s.T)


def _epi1_body(raw_ref, den_ref, st_ref, b_ref, o_ref):
    denb = jnp.dot(den_ref[...], st_ref[...],
                   preferred_element_type=jnp.float32)
    h = raw_ref[...] / (denb + 1e-16) + b_ref[...]
    o_ref[...] = jnp.where(h > 0, h, jnp.exp(jnp.minimum(h, 0.0)) - 1.0)


def _epi2_body(raw_ref, den_ref, st_ref, b_ref, o_ref):
    denb = jnp.dot(den_ref[...], st_ref[...],
                   preferred_element_type=jnp.float32)
    h = raw_ref[...] / (denb + 1e-16) + b_ref[...]
    m = jnp.max(h, axis=1, keepdims=True)
    o_ref[...] = h - m - jnp.log(jnp.sum(jnp.exp(h - m), axis=1,
                                         keepdims=True))


def _epilogue(body, raw, den, s, bias, bn=400):
    n, hc = raw.shape
    grid = (n // bn,)
    return pl.pallas_call(
        body,
        grid=grid,
        in_specs=[
            pl.BlockSpec((bn, hc), lambda i: (i, 0)),
            pl.BlockSpec((bn, HEADS), lambda i: (i, 0)),
            pl.BlockSpec((HEADS, hc), lambda i: (0, 0)),
            pl.BlockSpec((1, hc), lambda i: (0, 0)),
        ],
        out_specs=pl.BlockSpec((bn, hc), lambda i: (i, 0)),
        out_shape=jax.ShapeDtypeStruct((n, hc), jnp.float32),
        compiler_params=_PAR,
    )(raw, den, s.T, bias.reshape(1, hc))


def _conv(x, src, dst, W, att_l, att_r, bias, C, epi_body):
    n = x.shape[0]
    hc = HEADS * C
    s = jnp.repeat(jnp.eye(HEADS, dtype=jnp.float32), C, axis=0)
    al_mat = s * att_l.reshape(hc, 1)
    ar_mat = s * att_r.reshape(hc, 1)
    xl = _matmul(x, W)
    ex, msg = _edge_pass(xl[src], xl[dst], s, al_mat, ar_mat)
    denom = jax.ops.segment_sum(ex, dst, num_segments=n)
    raw = jax.ops.segment_sum(msg, dst, num_segments=n)
    return _epilogue(epi_body, raw, denom, s, bias)


def kernel(x, edge_index, W1, att_l1, att_r1, bias1, W2, att_l2, att_r2, bias2):
    loop = jnp.arange(N, dtype=edge_index.dtype)
    src = jnp.concatenate([edge_index[0], loop])
    dst = jnp.concatenate([edge_index[1], loop])
    h = _conv(x, src, dst, W1, att_l1, att_r1, bias1, 256, _epi1_body)
    return _conv(h, src, dst, W2, att_l2, att_r2, bias2, 128, _epi2_body)
